# baseline (speedup 1.0000x reference)
import functools

import jax
import jax.numpy as jnp
from jax.experimental import pallas as pl

_B, _N, _C, _A = 8, 20000, 80, 64
_NB = 10240


def _block_kernel(ct_ref, regt_ref, anch_ref, annb_ref, annd_hi_ref,
                  annd_lo_ref, cls_out, pos_out, reg_out):
    nb = pl.program_id(1)

    ax1 = anch_ref[0:1, :]
    ay1 = anch_ref[1:2, :]
    ax2 = anch_ref[2:3, :]
    ay2 = anch_ref[3:4, :]
    area_a = anch_ref[4:5, :]
    ann = annb_ref[0]
    bx1 = ann[:, 0:1]
    by1 = ann[:, 1:2]
    bx2 = ann[:, 2:3]
    by2 = ann[:, 3:4]
    area_b = ann[:, 4:5]

    iw = jnp.maximum(jnp.minimum(ax2, bx2) - jnp.maximum(ax1, bx1), 0.0)
    ih = jnp.maximum(jnp.minimum(ay2, by2) - jnp.maximum(ay1, by1), 0.0)
    inter = iw * ih
    iou = inter / (area_a + area_b - inter)

    iou_max = jnp.max(iou, axis=0, keepdims=True)
    srow = jax.lax.broadcasted_iota(jnp.int32, iou.shape, 0)
    first = jnp.min(jnp.where(iou == iou_max, srow, _A), axis=0, keepdims=True)

    oh16 = (srow == first).astype(jnp.float32).astype(jnp.bfloat16)
    dims = (((1,), (0,)), ((), ()))
    g = (jax.lax.dot_general(annd_hi_ref[0], oh16, dims,
                             preferred_element_type=jnp.float32)
         + jax.lax.dot_general(annd_lo_ref[0], oh16, dims,
                               preferred_element_type=jnp.float32))

    positive = (iou_max >= 0.5).astype(jnp.float32)
    valid = (iou_max >= 0.25).astype(jnp.float32)

    c = ct_ref[0].astype(jnp.float32)
    p = c * c * jnp.log(1.0 - c)
    s0 = jnp.sum(p, axis=0, keepdims=True)

    clane = jax.lax.broadcasted_iota(jnp.int32, c.shape, 0)
    gcl = g[4:5, :].astype(jnp.int32)
    ck = jnp.sum(jnp.where(clane == gcl, c, 0.0), axis=0, keepdims=True)
    f0k = -0.75 * ck * ck * jnp.log(1.0 - ck)
    f1k = -0.25 * (1.0 - ck) * (1.0 - ck) * jnp.log(ck)
    cls_row = valid * (-0.75 * s0) + positive * (f1k - f0k)

    t = (g[0:4, :] - anch_ref[8:12, :]) * anch_ref[12:16, :]
    d = jnp.abs(t - regt_ref[0])
    rl = jnp.where(d <= 1.0 / 9.0, 4.5 * d * d, d - 0.5 / 9.0)
    reg_row = jnp.sum(rl, axis=0, keepdims=True) * positive

    lane = jax.lax.broadcasted_iota(jnp.int32, (1, _NB), 1)
    inb = nb * _NB + lane < _N
    cls_partial = jnp.sum(jnp.where(inb, cls_row, 0.0))
    pos_partial = jnp.sum(jnp.where(inb, positive, 0.0))
    reg_partial = jnp.sum(jnp.where(inb, reg_row, 0.0))

    cvec = jnp.full((1, 1, 128), cls_partial, jnp.float32)
    pvec = jnp.full((1, 1, 128), pos_partial, jnp.float32)
    rvec = jnp.full((1, 1, 128), reg_partial, jnp.float32)

    @pl.when(nb == 0)
    def _init():
        cls_out[...] = cvec
        pos_out[...] = pvec
        reg_out[...] = rvec

    @pl.when(nb != 0)
    def _acc():
        cls_out[...] += cvec
        pos_out[...] += pvec
        reg_out[...] += rvec


@functools.partial(jax.jit, static_argnames=())
def kernel(classifications, regressions, anchors, annotations):
    ct = jnp.transpose(classifications, (0, 2, 1)).astype(jnp.bfloat16)
    regt = jnp.transpose(regressions, (0, 2, 1))

    a = anchors[0]
    aw = a[:, 2] - a[:, 0]
    ah = a[:, 3] - a[:, 1]
    one = jnp.ones_like(aw)
    anch = jnp.stack([
        a[:, 0], a[:, 1], a[:, 2], a[:, 3], aw * ah,
        one * 0.0, one * 0.0, one * 0.0,
        a[:, 0] + 0.5 * aw, a[:, 1] + 0.5 * ah, jnp.log(aw), jnp.log(ah),
        1.0 / aw, 1.0 / ah, one, one,
    ], axis=0)

    bx = annotations
    bw = bx[:, :, 2] - bx[:, :, 0]
    bh = bx[:, :, 3] - bx[:, :, 1]
    annb = jnp.concatenate([bx[:, :, :4], (bw * bh)[:, :, None]], axis=-1)
    zero = jnp.zeros_like(bw)
    annd = jnp.stack([
        bx[:, :, 0] + 0.5 * bw, bx[:, :, 1] + 0.5 * bh,
        jnp.log(bw), jnp.log(bh), bx[:, :, 4], zero, zero, zero,
    ], axis=1)
    annd_hi = annd.astype(jnp.bfloat16)
    annd_lo = (annd - annd_hi.astype(jnp.float32)).astype(jnp.bfloat16)

    nblk = pl.cdiv(_N, _NB)
    out_shapes = tuple(jax.ShapeDtypeStruct((_B, 1, 128), jnp.float32)
                       for _ in range(3))
    cls_s, pos_s, reg_s = pl.pallas_call(
        _block_kernel,
        grid=(_B, nblk),
        in_specs=[
            pl.BlockSpec((1, _C, _NB), lambda b, nb: (b, 0, nb)),
            pl.BlockSpec((1, 4, _NB), lambda b, nb: (b, 0, nb)),
            pl.BlockSpec((16, _NB), lambda b, nb: (0, nb)),
            pl.BlockSpec((1, _A, 5), lambda b, nb: (b, 0, 0)),
            pl.BlockSpec((1, 8, _A), lambda b, nb: (b, 0, 0)),
            pl.BlockSpec((1, 8, _A), lambda b, nb: (b, 0, 0)),
        ],
        out_specs=tuple(pl.BlockSpec((1, 1, 128), lambda b, nb: (b, 0, 0))
                        for _ in range(3)),
        out_shape=out_shapes,
    )(ct, regt, anch, annb, annd_hi, annd_lo)

    cs = cls_s[:, 0, 0]
    pc = pos_s[:, 0, 0]
    rs = reg_s[:, 0, 0]
    cls_j = cs / jnp.clip(pc, 0.01, None)
    reg_j = jnp.where(pc > 0.0, rs / jnp.maximum(pc * 4.0, 1.0), 0.0)
    return (jnp.mean(cls_j, keepdims=True), jnp.mean(reg_j, keepdims=True))


# speedup vs baseline: 1.3759x; 1.3759x over previous
import functools

import jax
import jax.numpy as jnp
from jax.experimental import pallas as pl

_B, _N, _C, _A = 8, 20000, 80, 64
_NB = 10240


def _block_kernel(ct_ref, regt_ref, anch_ref, annb_ref, annd_hi_ref,
                  annd_lo_ref, cls_out, pos_out, reg_out):
    nb = pl.program_id(1)

    ax1 = anch_ref[0:1, :]
    ay1 = anch_ref[1:2, :]
    ax2 = anch_ref[2:3, :]
    ay2 = anch_ref[3:4, :]
    area_a = anch_ref[4:5, :]
    ann = annb_ref[0]
    bx1 = ann[:, 0:1]
    by1 = ann[:, 1:2]
    bx2 = ann[:, 2:3]
    by2 = ann[:, 3:4]
    area_b = ann[:, 4:5]

    iw = jnp.maximum(jnp.minimum(ax2, bx2) - jnp.maximum(ax1, bx1), 0.0)
    ih = jnp.maximum(jnp.minimum(ay2, by2) - jnp.maximum(ay1, by1), 0.0)
    inter = iw * ih
    iou = inter / (area_a + area_b - inter)

    iou_max = jnp.max(iou, axis=0, keepdims=True)
    srow = jax.lax.broadcasted_iota(jnp.int32, iou.shape, 0)
    first = jnp.min(jnp.where(iou == iou_max, srow, _A), axis=0, keepdims=True)

    oh16 = (srow == first).astype(jnp.float32).astype(jnp.bfloat16)
    dims = (((1,), (0,)), ((), ()))
    g = (jax.lax.dot_general(annd_hi_ref[0], oh16, dims,
                             preferred_element_type=jnp.float32)
         + jax.lax.dot_general(annd_lo_ref[0], oh16, dims,
                               preferred_element_type=jnp.float32))

    positive = (iou_max >= 0.5).astype(jnp.float32)
    valid = (iou_max >= 0.25).astype(jnp.float32)

    c = ct_ref[0]
    p = c * c * jnp.log(1.0 - c)

    clane = jax.lax.broadcasted_iota(jnp.int32, c.shape, 0)
    gcl = g[4:5, :].astype(jnp.int32)
    ckp = jnp.where(clane == gcl, c, 0.0)

    ones16 = jnp.full((1, _C), 1, jnp.bfloat16)
    s0 = jax.lax.dot_general(ones16, p.astype(jnp.bfloat16), dims,
                             preferred_element_type=jnp.float32)
    ck = jax.lax.dot_general(ones16, ckp.astype(jnp.bfloat16), dims,
                             preferred_element_type=jnp.float32)
    f0k = -0.75 * ck * ck * jnp.log(1.0 - ck)
    f1k = -0.25 * (1.0 - ck) * (1.0 - ck) * jnp.log(ck)
    cls_row = valid * (-0.75 * s0) + positive * (f1k - f0k)

    t = (g[0:4, :] - anch_ref[8:12, :]) * anch_ref[12:16, :]
    d = jnp.abs(t - regt_ref[0])
    rl = jnp.where(d <= 1.0 / 9.0, 4.5 * d * d, d - 0.5 / 9.0)
    reg_row = jnp.sum(rl, axis=0, keepdims=True) * positive

    lane = jax.lax.broadcasted_iota(jnp.int32, (1, _NB), 1)
    inb = nb * _NB + lane < _N
    cls_partial = jnp.sum(jnp.where(inb, cls_row, 0.0))
    pos_partial = jnp.sum(jnp.where(inb, positive, 0.0))
    reg_partial = jnp.sum(jnp.where(inb, reg_row, 0.0))

    cvec = jnp.full((1, 1, 128), cls_partial, jnp.float32)
    pvec = jnp.full((1, 1, 128), pos_partial, jnp.float32)
    rvec = jnp.full((1, 1, 128), reg_partial, jnp.float32)

    @pl.when(nb == 0)
    def _init():
        cls_out[...] = cvec
        pos_out[...] = pvec
        reg_out[...] = rvec

    @pl.when(nb != 0)
    def _acc():
        cls_out[...] += cvec
        pos_out[...] += pvec
        reg_out[...] += rvec


@functools.partial(jax.jit, static_argnames=())
def kernel(classifications, regressions, anchors, annotations):
    ct = jnp.transpose(classifications, (0, 2, 1))
    regt = jnp.transpose(regressions, (0, 2, 1))

    a = anchors[0]
    aw = a[:, 2] - a[:, 0]
    ah = a[:, 3] - a[:, 1]
    one = jnp.ones_like(aw)
    anch = jnp.stack([
        a[:, 0], a[:, 1], a[:, 2], a[:, 3], aw * ah,
        one * 0.0, one * 0.0, one * 0.0,
        a[:, 0] + 0.5 * aw, a[:, 1] + 0.5 * ah, jnp.log(aw), jnp.log(ah),
        1.0 / aw, 1.0 / ah, one, one,
    ], axis=0)

    bx = annotations
    bw = bx[:, :, 2] - bx[:, :, 0]
    bh = bx[:, :, 3] - bx[:, :, 1]
    annb = jnp.concatenate([bx[:, :, :4], (bw * bh)[:, :, None]], axis=-1)
    zero = jnp.zeros_like(bw)
    annd = jnp.stack([
        bx[:, :, 0] + 0.5 * bw, bx[:, :, 1] + 0.5 * bh,
        jnp.log(bw), jnp.log(bh), bx[:, :, 4], zero, zero, zero,
    ], axis=1)
    annd_hi = annd.astype(jnp.bfloat16)
    annd_lo = (annd - annd_hi.astype(jnp.float32)).astype(jnp.bfloat16)

    nblk = pl.cdiv(_N, _NB)
    out_shapes = tuple(jax.ShapeDtypeStruct((_B, 1, 128), jnp.float32)
                       for _ in range(3))
    cls_s, pos_s, reg_s = pl.pallas_call(
        _block_kernel,
        grid=(_B, nblk),
        in_specs=[
            pl.BlockSpec((1, _C, _NB), lambda b, nb: (b, 0, nb)),
            pl.BlockSpec((1, 4, _NB), lambda b, nb: (b, 0, nb)),
            pl.BlockSpec((16, _NB), lambda b, nb: (0, nb)),
            pl.BlockSpec((1, _A, 5), lambda b, nb: (b, 0, 0)),
            pl.BlockSpec((1, 8, _A), lambda b, nb: (b, 0, 0)),
            pl.BlockSpec((1, 8, _A), lambda b, nb: (b, 0, 0)),
        ],
        out_specs=tuple(pl.BlockSpec((1, 1, 128), lambda b, nb: (b, 0, 0))
                        for _ in range(3)),
        out_shape=out_shapes,
    )(ct, regt, anch, annb, annd_hi, annd_lo)

    cs = cls_s[:, 0, 0]
    pc = pos_s[:, 0, 0]
    rs = reg_s[:, 0, 0]
    cls_j = cs / jnp.clip(pc, 0.01, None)
    reg_j = jnp.where(pc > 0.0, rs / jnp.maximum(pc * 4.0, 1.0), 0.0)
    return (jnp.mean(cls_j, keepdims=True), jnp.mean(reg_j, keepdims=True))
